# baseline (speedup 1.0000x reference)
import functools

import jax
import jax.numpy as jnp
from jax import lax
from jax.experimental import pallas as pl
from jax.experimental.pallas import tpu as pltpu
from jax.experimental.pallas import tpu_sc as plsc

B = 16384
V = 100000
D = 128

NC = 2
NS = 16
L = 16
NW = NC * NS
PER_W = B // NW
C = 256
N_CHUNKS = PER_W // C


def _neg_sqrt(ssq):
    x = jnp.maximum(ssq, jnp.float32(1e-35))
    bits = lax.bitcast_convert_type(x, jnp.int32)
    y = lax.bitcast_convert_type(
        jnp.int32(0x5F3759DF) - lax.shift_right_logical(bits, 1), jnp.float32)
    for _ in range(3):
        y = y * (jnp.float32(1.5) - jnp.float32(0.5) * x * y * y)
    return -(x * y)


def _shuffle(x, idx):
    return lax.gather(
        x, idx[:, None],
        lax.GatherDimensionNumbers(
            offset_dims=(), collapsed_slice_dims=(0,), start_index_map=(0,)),
        slice_sizes=(1,), mode=lax.GatherScatterMode.PROMISE_IN_BOUNDS)


def _body(mention_hbm, h_hbm, t_hbm, table_hbm, out_hbm,
          idxh_v, idxt_v, h_rows, t_rows, m_rows, acc_buf, out_v, sem):
    wid = lax.axis_index("s") * NC + lax.axis_index("c")
    base = wid * PER_W

    row_iota = lax.iota(jnp.int32, L)

    for c in range(N_CHUNKS):
        off = base + c * C
        pltpu.sync_copy(h_hbm.at[pl.ds(off, C)], idxh_v)
        pltpu.sync_copy(t_hbm.at[pl.ds(off, C)], idxt_v)
        cp_h = pltpu.async_copy(table_hbm.at[idxh_v], h_rows, sem)
        cp_t = pltpu.async_copy(table_hbm.at[idxt_v], t_rows, sem)
        cp_m = pltpu.async_copy(mention_hbm.at[pl.ds(off, C)], m_rows, sem)
        cp_h.wait()
        cp_t.wait()
        cp_m.wait()

        def group(g, carry, _c=c):
            ssq = jnp.zeros((L,), jnp.float32)
            for r in range(L):
                row = g * L + r
                acc = jnp.zeros((L,), jnp.float32)
                for k in range(D // L):
                    hv = h_rows[row, pl.ds(k * L, L)]
                    mv = m_rows[row, pl.ds(k * L, L)]
                    tv = t_rows[row, pl.ds(k * L, L)]
                    d = (hv + mv) - tv
                    acc = acc + d * d
                for sh in (8, 4, 2, 1):
                    acc = acc + _shuffle(acc, (row_iota + sh) % L)
                ssq = jnp.where(row_iota == r, acc, ssq)
            out_v[pl.ds(_c * C + g * L, L)] = _neg_sqrt(ssq)
            return carry

        lax.fori_loop(0, C // L, group, 0)

    pltpu.sync_copy(out_v, out_hbm.at[pl.ds(base, PER_W)])


_mesh = plsc.VectorSubcoreMesh(core_axis_name="c", subcore_axis_name="s")

_triplet = functools.partial(
    pl.kernel,
    mesh=_mesh,
    out_type=jax.ShapeDtypeStruct((B,), jnp.float32),
    scratch_types=[
        pltpu.VMEM((C,), jnp.int32),
        pltpu.VMEM((C,), jnp.int32),
        pltpu.VMEM((C, D), jnp.float32),
        pltpu.VMEM((C, D), jnp.float32),
        pltpu.VMEM((C, D), jnp.float32),
        pltpu.VMEM((L,), jnp.float32),
        pltpu.VMEM((PER_W,), jnp.float32),
        pltpu.SemaphoreType.DMA,
    ],
)(_body)


def kernel(mention, h, t, emb_table):
    assert mention.shape == (B, D) and emb_table.shape == (V, D)
    assert h.shape == (B,) and t.shape == (B,)
    return _triplet(mention, h, t, emb_table)


# speedup vs baseline: 1.1517x; 1.1517x over previous
import functools

import jax
import jax.numpy as jnp
from jax import lax
from jax.experimental import pallas as pl
from jax.experimental.pallas import tpu as pltpu
from jax.experimental.pallas import tpu_sc as plsc

B = 16384
V = 100000
D = 128

NC = 2
NS = 16
L = 16
NW = NC * NS
PER_W = B // NW
C = 128
N_CHUNKS = PER_W // C
NBUF = 2


def _bitrev4(x):
    return ((x & 1) << 3) | ((x & 2) << 1) | ((x & 4) >> 1) | ((x & 8) >> 3)


def _shuffle(x, idx):
    return lax.gather(
        x, idx[:, None],
        lax.GatherDimensionNumbers(
            offset_dims=(), collapsed_slice_dims=(0,), start_index_map=(0,)),
        slice_sizes=(1,), mode=lax.GatherScatterMode.PROMISE_IN_BOUNDS)


def _neg_sqrt(ssq):
    x = jnp.maximum(ssq, jnp.float32(1e-35))
    bits = lax.bitcast_convert_type(x, jnp.int32)
    y = lax.bitcast_convert_type(
        jnp.int32(0x5F3759DF) - lax.shift_right_logical(bits, 1), jnp.float32)
    for _ in range(3):
        y = y * (jnp.float32(1.5) - jnp.float32(0.5) * x * y * y)
    return -(x * y)


def _body(mention_hbm, h_hbm, t_hbm, table_hbm, out_hbm,
          idxh_v, idxt_v, h_rows, t_rows, m_rows, out_v, sems):
    wid = lax.axis_index("s") * NC + lax.axis_index("c")
    base = wid * PER_W

    lane = lax.iota(jnp.int32, L)

    pltpu.sync_copy(h_hbm.at[pl.ds(base, PER_W)], idxh_v)
    pltpu.sync_copy(t_hbm.at[pl.ds(base, PER_W)], idxt_v)

    def start(c):
        buf = c % NBUF
        return (
            pltpu.async_copy(table_hbm.at[idxh_v.at[pl.ds(c * C, C)]],
                             h_rows.at[buf], sems.at[buf, 0]),
            pltpu.async_copy(table_hbm.at[idxt_v.at[pl.ds(c * C, C)]],
                             t_rows.at[buf], sems.at[buf, 1]),
            pltpu.async_copy(mention_hbm.at[pl.ds(base + c * C, C)],
                             m_rows.at[buf], sems.at[buf, 2]),
        )

    pending = start(0)
    for c in range(N_CHUNKS):
        for cp in pending:
            cp.wait()
        if c + 1 < N_CHUNKS:
            pending = start(c + 1)
        buf = c % NBUF

        def group(g, carry, _buf=buf, _c=c):
            hb, tb, mb = h_rows.at[_buf], t_rows.at[_buf], m_rows.at[_buf]
            ssq = jnp.zeros((L,), jnp.float32)
            for r in range(L):
                row = g * L + r
                acc = None
                for k in range(D // L):
                    hv = hb[row, pl.ds(k * L, L)]
                    mv = mb[row, pl.ds(k * L, L)]
                    tv = tb[row, pl.ds(k * L, L)]
                    d = (hv + mv) - tv
                    acc = d * d if acc is None else acc + d * d
                for sh in (8, 4, 2, 1):
                    acc = acc + _shuffle(acc, (lane + sh) % L)
                ssq = jnp.where(lane == r, acc, ssq)
            out_v[pl.ds(_c * C + g * L, L)] = _neg_sqrt(ssq)
            return carry

        lax.fori_loop(0, C // L, group, 0)

    pltpu.sync_copy(out_v, out_hbm.at[pl.ds(base, PER_W)])


_mesh = plsc.VectorSubcoreMesh(core_axis_name="c", subcore_axis_name="s")

_triplet = functools.partial(
    pl.kernel,
    mesh=_mesh,
    out_type=jax.ShapeDtypeStruct((B,), jnp.float32),
    scratch_types=[
        pltpu.VMEM((PER_W,), jnp.int32),
        pltpu.VMEM((PER_W,), jnp.int32),
        pltpu.VMEM((NBUF, C, D), jnp.float32),
        pltpu.VMEM((NBUF, C, D), jnp.float32),
        pltpu.VMEM((NBUF, C, D), jnp.float32),
        pltpu.VMEM((PER_W,), jnp.float32),
        pltpu.SemaphoreType.DMA((NBUF, 3)),
    ],
)(_body)


def kernel(mention, h, t, emb_table):
    assert mention.shape == (B, D) and emb_table.shape == (V, D)
    assert h.shape == (B,) and t.shape == (B,)
    return _triplet(mention, h, t, emb_table)


# speedup vs baseline: 1.2307x; 1.0686x over previous
import functools

import jax
import jax.numpy as jnp
from jax import lax
from jax.experimental import pallas as pl
from jax.experimental.pallas import tpu as pltpu
from jax.experimental.pallas import tpu_sc as plsc

B = 16384
V = 100000
D = 128

NC = 2
NS = 16
L = 16
NW = NC * NS
PER_W = B // NW
C = 128
N_CHUNKS = PER_W // C
NBUF = 2


def _bitrev4(x):
    return ((x & 1) << 3) | ((x & 2) << 1) | ((x & 4) >> 1) | ((x & 8) >> 3)


def _shuffle(x, idx):
    return lax.gather(
        x, idx[:, None],
        lax.GatherDimensionNumbers(
            offset_dims=(), collapsed_slice_dims=(0,), start_index_map=(0,)),
        slice_sizes=(1,), mode=lax.GatherScatterMode.PROMISE_IN_BOUNDS)


def _neg_sqrt(ssq):
    x = jnp.maximum(ssq, jnp.float32(1e-35))
    bits = lax.bitcast_convert_type(x, jnp.int32)
    y = lax.bitcast_convert_type(
        jnp.int32(0x5F3759DF) - lax.shift_right_logical(bits, 1), jnp.float32)
    for _ in range(3):
        y = y * (jnp.float32(1.5) - jnp.float32(0.5) * x * y * y)
    return -(x * y)


def _body(mention_hbm, h_hbm, t_hbm, table_hbm, out_hbm,
          idxh_v, idxt_v, h_rows, t_rows, m_rows, out_v, sems):
    wid = lax.axis_index("s") * NC + lax.axis_index("c")
    base = wid * PER_W

    lane = lax.iota(jnp.int32, L)

    pltpu.sync_copy(h_hbm.at[pl.ds(base, PER_W)], idxh_v)
    pltpu.sync_copy(t_hbm.at[pl.ds(base, PER_W)], idxt_v)

    def start(c):
        buf = c % NBUF
        return (
            pltpu.async_copy(table_hbm.at[idxh_v.at[pl.ds(c * C, C)]],
                             h_rows.at[buf], sems.at[buf, 0]),
            pltpu.async_copy(table_hbm.at[idxt_v.at[pl.ds(c * C, C)]],
                             t_rows.at[buf], sems.at[buf, 1]),
            pltpu.async_copy(mention_hbm.at[pl.ds(base + c * C, C)],
                             m_rows.at[buf], sems.at[buf, 2]),
        )

    pending = start(0)
    for c in range(N_CHUNKS):
        for cp in pending:
            cp.wait()
        if c + 1 < N_CHUNKS:
            pending = start(c + 1)
        buf = c % NBUF

        hb, tb, mb = h_rows.at[buf], t_rows.at[buf], m_rows.at[buf]
        for gg in range(C // L):

            def row_body(r, ssq, _gg=gg):
                row = _gg * L + r
                acc = None
                for k in range(D // L):
                    hv = hb[row, pl.ds(k * L, L)]
                    mv = mb[row, pl.ds(k * L, L)]
                    tv = tb[row, pl.ds(k * L, L)]
                    d = (hv + mv) - tv
                    acc = d * d if acc is None else acc + d * d
                for sh in (8, 4, 2, 1):
                    acc = acc + _shuffle(acc, (lane + sh) % L)
                return jnp.where(lane == r, acc, ssq)

            ssq = lax.fori_loop(0, L, row_body,
                                jnp.zeros((L,), jnp.float32), unroll=2)
            out_v[pl.ds(c * C + gg * L, L)] = _neg_sqrt(ssq)

    pltpu.sync_copy(out_v, out_hbm.at[pl.ds(base, PER_W)])


_mesh = plsc.VectorSubcoreMesh(core_axis_name="c", subcore_axis_name="s")

_triplet = functools.partial(
    pl.kernel,
    mesh=_mesh,
    out_type=jax.ShapeDtypeStruct((B,), jnp.float32),
    scratch_types=[
        pltpu.VMEM((PER_W,), jnp.int32),
        pltpu.VMEM((PER_W,), jnp.int32),
        pltpu.VMEM((NBUF, C, D), jnp.float32),
        pltpu.VMEM((NBUF, C, D), jnp.float32),
        pltpu.VMEM((NBUF, C, D), jnp.float32),
        pltpu.VMEM((PER_W,), jnp.float32),
        pltpu.SemaphoreType.DMA((NBUF, 3)),
    ],
)(_body)


def kernel(mention, h, t, emb_table):
    assert mention.shape == (B, D) and emb_table.shape == (V, D)
    assert h.shape == (B,) and t.shape == (B,)
    return _triplet(mention, h, t, emb_table)


# speedup vs baseline: 1.2932x; 1.0508x over previous
import functools

import jax
import jax.numpy as jnp
from jax import lax
from jax.experimental import pallas as pl
from jax.experimental.pallas import tpu as pltpu
from jax.experimental.pallas import tpu_sc as plsc

B = 16384
V = 100000
D = 128

NC = 2
NS = 16
L = 16
NW = NC * NS
PER_W = B // NW
C = 64
N_CHUNKS = PER_W // C
NBUF = 4


def _bitrev4(x):
    return ((x & 1) << 3) | ((x & 2) << 1) | ((x & 4) >> 1) | ((x & 8) >> 3)


def _shuffle(x, idx):
    return lax.gather(
        x, idx[:, None],
        lax.GatherDimensionNumbers(
            offset_dims=(), collapsed_slice_dims=(0,), start_index_map=(0,)),
        slice_sizes=(1,), mode=lax.GatherScatterMode.PROMISE_IN_BOUNDS)


def _neg_sqrt(ssq):
    x = jnp.maximum(ssq, jnp.float32(1e-35))
    bits = lax.bitcast_convert_type(x, jnp.int32)
    y = lax.bitcast_convert_type(
        jnp.int32(0x5F3759DF) - lax.shift_right_logical(bits, 1), jnp.float32)
    for _ in range(3):
        y = y * (jnp.float32(1.5) - jnp.float32(0.5) * x * y * y)
    return -(x * y)


def _body(mention_hbm, h_hbm, t_hbm, table_hbm, out_hbm,
          idxh_v, idxt_v, h_rows, t_rows, m_rows, out_v, sems):
    wid = lax.axis_index("s") * NC + lax.axis_index("c")
    base = wid * PER_W

    lane = lax.iota(jnp.int32, L)

    cp_ih = pltpu.async_copy(h_hbm.at[pl.ds(base, PER_W)], idxh_v,
                             sems.at[0, 0])
    cp_it = pltpu.async_copy(t_hbm.at[pl.ds(base, PER_W)], idxt_v,
                             sems.at[0, 1])
    cp_ih.wait()
    cp_it.wait()

    def start(c):
        buf = c % NBUF
        return (
            pltpu.async_copy(table_hbm.at[idxh_v.at[pl.ds(c * C, C)]],
                             h_rows.at[buf], sems.at[buf, 0]),
            pltpu.async_copy(table_hbm.at[idxt_v.at[pl.ds(c * C, C)]],
                             t_rows.at[buf], sems.at[buf, 1]),
            pltpu.async_copy(mention_hbm.at[pl.ds(base + c * C, C)],
                             m_rows.at[buf], sems.at[buf, 2]),
        )

    pending = [start(c) for c in range(NBUF - 1)]
    for c in range(N_CHUNKS):
        for cp in pending.pop(0):
            cp.wait()
        if c + NBUF - 1 < N_CHUNKS:
            pending.append(start(c + NBUF - 1))
        buf = c % NBUF

        hb, tb, mb = h_rows.at[buf], t_rows.at[buf], m_rows.at[buf]
        for gg in range(C // L):

            def row_body(r, ssq, _gg=gg):
                row = _gg * L + r
                acc = None
                for k in range(D // L):
                    hv = hb[row, pl.ds(k * L, L)]
                    mv = mb[row, pl.ds(k * L, L)]
                    tv = tb[row, pl.ds(k * L, L)]
                    d = (hv + mv) - tv
                    acc = d * d if acc is None else acc + d * d
                for sh in (8, 4, 2, 1):
                    acc = acc + _shuffle(acc, (lane + sh) % L)
                return jnp.where(lane == r, acc, ssq)

            ssq = lax.fori_loop(0, L, row_body,
                                jnp.zeros((L,), jnp.float32), unroll=2)
            out_v[pl.ds(c * C + gg * L, L)] = _neg_sqrt(ssq)

    pltpu.sync_copy(out_v, out_hbm.at[pl.ds(base, PER_W)])


_mesh = plsc.VectorSubcoreMesh(core_axis_name="c", subcore_axis_name="s")

_triplet = functools.partial(
    pl.kernel,
    mesh=_mesh,
    out_type=jax.ShapeDtypeStruct((B,), jnp.float32),
    scratch_types=[
        pltpu.VMEM((PER_W,), jnp.int32),
        pltpu.VMEM((PER_W,), jnp.int32),
        pltpu.VMEM((NBUF, C, D), jnp.float32),
        pltpu.VMEM((NBUF, C, D), jnp.float32),
        pltpu.VMEM((NBUF, C, D), jnp.float32),
        pltpu.VMEM((PER_W,), jnp.float32),
        pltpu.SemaphoreType.DMA((NBUF, 3)),
    ],
)(_body)


def kernel(mention, h, t, emb_table):
    assert mention.shape == (B, D) and emb_table.shape == (V, D)
    assert h.shape == (B,) and t.shape == (B,)
    return _triplet(mention, h, t, emb_table)


# speedup vs baseline: 1.7055x; 1.3188x over previous
import functools

import jax
import jax.numpy as jnp
from jax import lax
from jax.experimental import pallas as pl
from jax.experimental.pallas import tpu as pltpu
from jax.experimental.pallas import tpu_sc as plsc

B = 16384
V = 100000
D = 128

NC = 2
NS = 16
L = 16
NW = NC * NS
PER_W = B // NW
C = 64
N_CHUNKS = PER_W // C
NBUF = 4


def _shuffle(x, idx):
    return lax.gather(
        x, idx[:, None],
        lax.GatherDimensionNumbers(
            offset_dims=(), collapsed_slice_dims=(0,), start_index_map=(0,)),
        slice_sizes=(1,), mode=lax.GatherScatterMode.PROMISE_IN_BOUNDS)


def _neg_sqrt(ssq):
    x = jnp.maximum(ssq, jnp.float32(1e-35))
    bits = lax.bitcast_convert_type(x, jnp.int32)
    y = lax.bitcast_convert_type(
        jnp.int32(0x5F3759DF) - lax.shift_right_logical(bits, 1), jnp.float32)
    for _ in range(3):
        y = y * (jnp.float32(1.5) - jnp.float32(0.5) * x * y * y)
    return -(x * y)


def _body(mention_hbm, h_hbm, t_hbm, table_hbm, out_hbm,
          idxh_v, idxt_v, h_rows, t_rows, m_rows, out_v, sems):
    wid = lax.axis_index("s") * NC + lax.axis_index("c")
    base = wid * PER_W

    lane = lax.iota(jnp.int32, L)

    cp_ih = pltpu.async_copy(h_hbm.at[pl.ds(base, PER_W)], idxh_v,
                             sems.at[0, 0])
    cp_it = pltpu.async_copy(t_hbm.at[pl.ds(base, PER_W)], idxt_v,
                             sems.at[0, 1])
    cp_ih.wait()
    cp_it.wait()

    def dma_trio(c, b):
        return (
            pltpu.make_async_copy(table_hbm.at[idxh_v.at[pl.ds(c * C, C)]],
                                  h_rows.at[b], sems.at[b, 0]),
            pltpu.make_async_copy(table_hbm.at[idxt_v.at[pl.ds(c * C, C)]],
                                  t_rows.at[b], sems.at[b, 1]),
            pltpu.make_async_copy(mention_hbm.at[pl.ds(base + c * C, C)],
                                  m_rows.at[b], sems.at[b, 2]),
        )

    for p in range(NBUF - 1):
        for cp in dma_trio(p, p):
            cp.start()

    def chunk_body(c, carry):
        b = lax.rem(c, NBUF)
        for cp in dma_trio(c, b):
            cp.wait()

        @pl.when(c <= N_CHUNKS - NBUF)
        def _prefetch():
            cc = c + NBUF - 1
            for cp in dma_trio(cc, lax.rem(cc, NBUF)):
                cp.start()

        def group_body(gg, carry2):
            def row_body(r, ssq):
                row = gg * L + r
                acc = None
                for k in range(D // L):
                    hv = h_rows[b, row, pl.ds(k * L, L)]
                    mv = m_rows[b, row, pl.ds(k * L, L)]
                    tv = t_rows[b, row, pl.ds(k * L, L)]
                    d = (hv + mv) - tv
                    acc = d * d if acc is None else acc + d * d
                for sh in (8, 4, 2, 1):
                    acc = acc + _shuffle(acc, (lane + sh) % L)
                return jnp.where(lane == r, acc, ssq)

            ssq = lax.fori_loop(0, L, row_body,
                                jnp.zeros((L,), jnp.float32), unroll=2)
            out_v[pl.ds(c * C + gg * L, L)] = _neg_sqrt(ssq)
            return carry2

        lax.fori_loop(0, C // L, group_body, 0)
        return carry

    lax.fori_loop(0, N_CHUNKS, chunk_body, 0)

    pltpu.sync_copy(out_v, out_hbm.at[pl.ds(base, PER_W)])


_mesh = plsc.VectorSubcoreMesh(core_axis_name="c", subcore_axis_name="s")

_triplet = functools.partial(
    pl.kernel,
    mesh=_mesh,
    out_type=jax.ShapeDtypeStruct((B,), jnp.float32),
    scratch_types=[
        pltpu.VMEM((PER_W,), jnp.int32),
        pltpu.VMEM((PER_W,), jnp.int32),
        pltpu.VMEM((NBUF, C, D), jnp.float32),
        pltpu.VMEM((NBUF, C, D), jnp.float32),
        pltpu.VMEM((NBUF, C, D), jnp.float32),
        pltpu.VMEM((PER_W,), jnp.float32),
        pltpu.SemaphoreType.DMA((NBUF, 3)),
    ],
)(_body)


def kernel(mention, h, t, emb_table):
    assert mention.shape == (B, D) and emb_table.shape == (V, D)
    assert h.shape == (B,) and t.shape == (B,)
    return _triplet(mention, h, t, emb_table)
